# Initial kernel scaffold; baseline (speedup 1.0000x reference)
import jax
import jax.numpy as jnp
from jax.experimental import pallas as pl


def kernel(indexes, x):
    raise NotImplementedError("write your pallas kernel here")


# speedup vs baseline: 3.6119x; 3.6119x over previous
import functools

import jax
import jax.numpy as jnp
from jax import lax
from jax.experimental import pallas as pl
from jax.experimental.pallas import tpu as pltpu
from jax.experimental.pallas import tpu_sc as plsc

B = 320000
V = 10000
D = 128
NC = 2
NS = 16
NW = NC * NS
PER_W = B // NW
CHUNK = 80
NCHUNK = PER_W // CHUNK


def _gather_body(idx_hbm, x_hbm, out_hbm, idx_v, rows_v, sem):
    wid = lax.axis_index("s") * NC + lax.axis_index("c")
    base = wid * PER_W
    pltpu.sync_copy(idx_hbm.at[pl.ds(base, PER_W)], idx_v)

    def body(j, carry):
        off = pl.multiple_of(j * CHUNK, CHUNK)
        pltpu.async_copy(x_hbm.at[idx_v.at[pl.ds(off, CHUNK)]], rows_v, sem).wait()
        pltpu.sync_copy(rows_v, out_hbm.at[pl.ds(base + off, CHUNK)])
        return carry

    lax.fori_loop(0, NCHUNK, body, 0, unroll=False)


@jax.jit
def _gather(indexes, x):
    mesh = plsc.VectorSubcoreMesh(core_axis_name="c", subcore_axis_name="s")
    kfn = functools.partial(
        pl.kernel,
        mesh=mesh,
        out_type=jax.ShapeDtypeStruct((B, D), jnp.float32),
        scratch_types=[
            pltpu.VMEM((PER_W,), jnp.int32),
            pltpu.VMEM((CHUNK, D), jnp.float32),
            pltpu.SemaphoreType.DMA,
        ],
    )(_gather_body)
    return kfn(indexes, x)


def kernel(indexes, x):
    return _gather(indexes, x)


# speedup vs baseline: 5.5306x; 1.5312x over previous
import functools

import jax
import jax.numpy as jnp
from jax import lax
from jax.experimental import pallas as pl
from jax.experimental.pallas import tpu as pltpu
from jax.experimental.pallas import tpu_sc as plsc

B = 320000
V = 10000
D = 128
NC = 2
NS = 16
NW = NC * NS
PER_W = B // NW
CHUNK = 80
NCHUNK = PER_W // CHUNK


def _gather_body(idx_hbm, x_hbm, out_hbm, idx_v, rows0, rows1, sem0, sem1):
    wid = lax.axis_index("s") * NC + lax.axis_index("c")
    base = wid * PER_W
    pltpu.sync_copy(idx_hbm.at[pl.ds(base, PER_W)], idx_v)

    def start_gather(j, buf, sem):
        off = pl.multiple_of(j * CHUNK, CHUNK)
        pltpu.async_copy(x_hbm.at[idx_v.at[pl.ds(off, CHUNK)]], buf, sem)

    def finish_chunk(j, buf, sem):
        off = pl.multiple_of(j * CHUNK, CHUNK)
        pltpu.make_async_copy(x_hbm.at[idx_v.at[pl.ds(off, CHUNK)]], buf, sem).wait()
        pltpu.sync_copy(buf, out_hbm.at[pl.ds(base + off, CHUNK)])

    start_gather(0, rows0, sem0)

    def body(j, carry):
        def even():
            start_gather(j + 1, rows1, sem1)
            finish_chunk(j, rows0, sem0)

        def odd():
            start_gather(j + 1, rows0, sem0)
            finish_chunk(j, rows1, sem1)

        pl.when(j % 2 == 0)(even)
        pl.when(j % 2 == 1)(odd)
        return carry

    lax.fori_loop(0, NCHUNK - 1, body, 0, unroll=False)
    finish_chunk(NCHUNK - 1, rows0, sem0)


@jax.jit
def _gather(indexes, x):
    mesh = plsc.VectorSubcoreMesh(core_axis_name="c", subcore_axis_name="s")
    kfn = functools.partial(
        pl.kernel,
        mesh=mesh,
        out_type=jax.ShapeDtypeStruct((B, D), jnp.float32),
        scratch_types=[
            pltpu.VMEM((PER_W,), jnp.int32),
            pltpu.VMEM((CHUNK, D), jnp.float32),
            pltpu.VMEM((CHUNK, D), jnp.float32),
            pltpu.SemaphoreType.DMA,
            pltpu.SemaphoreType.DMA,
        ],
    )(_gather_body)
    return kfn(indexes, x)


def kernel(indexes, x):
    return _gather(indexes, x)


# speedup vs baseline: 6.0821x; 1.0997x over previous
import functools

import jax
import jax.numpy as jnp
from jax import lax
from jax.experimental import pallas as pl
from jax.experimental.pallas import tpu as pltpu
from jax.experimental.pallas import tpu_sc as plsc

B = 320000
V = 10000
D = 128
NC = 2
NS = 16
NW = NC * NS
PER_W = B // NW
CHUNK = 80
NCHUNK = PER_W // CHUNK


NBUF = 5
NGROUP = NCHUNK // NBUF


def _gather_body(idx_hbm, x_hbm, out_hbm, idx_v, *bufs_and_sems):
    bufs = bufs_and_sems[:NBUF]
    gsem = bufs_and_sems[NBUF:2 * NBUF]
    ssem = bufs_and_sems[2 * NBUF:3 * NBUF]
    wid = lax.axis_index("s") * NC + lax.axis_index("c")
    base = wid * PER_W
    pltpu.sync_copy(idx_hbm.at[pl.ds(base, PER_W)], idx_v)

    def start_gather(j, b):
        off = pl.multiple_of(j * CHUNK, CHUNK)
        pltpu.async_copy(x_hbm.at[idx_v.at[pl.ds(off, CHUNK)]], bufs[b], gsem[b])

    def wait_gather(j, b):
        off = pl.multiple_of(j * CHUNK, CHUNK)
        pltpu.make_async_copy(
            x_hbm.at[idx_v.at[pl.ds(off, CHUNK)]], bufs[b], gsem[b]).wait()

    def start_store(j, b):
        off = pl.multiple_of(j * CHUNK, CHUNK)
        pltpu.async_copy(bufs[b], out_hbm.at[pl.ds(base + off, CHUNK)], ssem[b])

    def wait_store(j, b):
        off = pl.multiple_of(j * CHUNK, CHUNK)
        pltpu.make_async_copy(
            bufs[b], out_hbm.at[pl.ds(base + off, CHUNK)], ssem[b]).wait()

    start_gather(0, 0)
    start_gather(1, 1)

    def group(g, carry):
        for b in range(NBUF):
            j = g * NBUF + b
            b2 = (b + 2) % NBUF
            if b >= 3:
                wait_store(j - 3, b2)
            else:
                pl.when(g >= 1)(lambda: wait_store(j - 3, b2))
            if b >= 3:
                pl.when(g < NGROUP - 1)(lambda: start_gather(j + 2, b2))
            else:
                start_gather(j + 2, b2)
            wait_gather(j, b)
            start_store(j, b)
        return carry

    lax.fori_loop(0, NGROUP, group, 0, unroll=False)
    wait_store(NCHUNK - 3, (NCHUNK - 3) % NBUF)
    wait_store(NCHUNK - 2, (NCHUNK - 2) % NBUF)
    wait_store(NCHUNK - 1, (NCHUNK - 1) % NBUF)


@jax.jit
def _gather(indexes, x):
    mesh = plsc.VectorSubcoreMesh(core_axis_name="c", subcore_axis_name="s")
    kfn = functools.partial(
        pl.kernel,
        mesh=mesh,
        out_type=jax.ShapeDtypeStruct((B, D), jnp.float32),
        scratch_types=(
            [pltpu.VMEM((PER_W,), jnp.int32)]
            + [pltpu.VMEM((CHUNK, D), jnp.float32) for _ in range(NBUF)]
            + [pltpu.SemaphoreType.DMA for _ in range(2 * NBUF)]
        ),
    )(_gather_body)
    return kfn(indexes, x)


def kernel(indexes, x):
    return _gather(indexes, x)
